# baseline (speedup 1.0000x reference)
import functools

import jax
import jax.numpy as jnp
from jax import lax
from jax.experimental import pallas as pl
from jax.experimental.pallas import tpu as pltpu, tpu_sc as plsc

N = 10000
D = 128
E = 320000

NC = 2
NS = 16
NW = NC * NS

CH = 128
NCHUNK = 80
E_PAD = NW * NCHUNK * CH
ACC_N = 10112
RPT = ACC_N // NS

_mesh = plsc.VectorSubcoreMesh(core_axis_name="c", subcore_axis_name="s",
                               num_cores=NC, num_subcores=NS)


def _deg_body(dst_hbm, ones_hbm, zrows_hbm, deg_out, dstidx, onesbuf, deg_sh):
    cid = lax.axis_index("c")
    sid = lax.axis_index("s")
    wid = cid * NS + sid
    base_r = sid * RPT
    pltpu.sync_copy(zrows_hbm, deg_sh.at[pl.ds(base_r, RPT)])
    pltpu.sync_copy(ones_hbm, onesbuf)
    pltpu.sync_copy(dst_hbm.at[wid], dstidx)
    plsc.subcore_barrier()

    def chunk(i, carry):
        pltpu.sync_copy(onesbuf, deg_sh.at[dstidx.at[i]], add=True)
        return carry

    lax.fori_loop(0, NCHUNK, chunk, 0)
    plsc.subcore_barrier()
    pltpu.sync_copy(deg_sh.at[pl.ds(base_r, RPT)],
                    deg_out.at[cid, pl.ds(base_r, RPT)])


WCH = 16
WPAIR = WCH // 2
SLOW_CORE = 1
C_SLOW = 80
C_FAST = 2 * NCHUNK - C_SLOW


HCH = CH // 2


def _agg_body(y_hbm, src_hbm, dst_hbm, zrows_hbm, acc_out,
              srcidx, dstidx, rows0, rows1, acc_sh,
              sem0a, sem0b, sem1a, sem1b):
    cid = lax.axis_index("c")
    sid = lax.axis_index("s")
    base_r = sid * RPT
    base_chunk = jnp.where(cid == SLOW_CORE, sid * C_SLOW,
                           NS * C_SLOW + sid * C_FAST)
    nwin = jnp.where(cid == SLOW_CORE, C_SLOW // WCH, C_FAST // WCH)
    pltpu.sync_copy(zrows_hbm, acc_sh.at[pl.ds(base_r, RPT)])
    plsc.subcore_barrier()

    def gath(i, rows, sa, sb):
        pltpu.async_copy(y_hbm.at[srcidx.at[i, pl.ds(0, HCH)]],
                         rows.at[pl.ds(0, HCH)], sa)
        pltpu.async_copy(y_hbm.at[srcidx.at[i, pl.ds(HCH, HCH)]],
                         rows.at[pl.ds(HCH, HCH)], sb)

    def gwait(i, rows, sa, sb):
        pltpu.make_async_copy(y_hbm.at[srcidx.at[i, pl.ds(0, HCH)]],
                              rows.at[pl.ds(0, HCH)], sa).wait()
        pltpu.make_async_copy(y_hbm.at[srcidx.at[i, pl.ds(HCH, HCH)]],
                              rows.at[pl.ds(HCH, HCH)], sb).wait()

    def window(w, carry):
        cb = base_chunk + w * WCH
        pltpu.sync_copy(src_hbm.at[pl.ds(cb, WCH)], srcidx)
        pltpu.sync_copy(dst_hbm.at[pl.ds(cb, WCH)], dstidx)
        gath(0, rows0, sem0a, sem0b)

        def pair(j, c):
            i0 = 2 * j
            gath(i0 + 1, rows1, sem1a, sem1b)
            gwait(i0, rows0, sem0a, sem0b)
            pltpu.sync_copy(rows0, acc_sh.at[dstidx.at[i0]], add=True)

            @pl.when(j < WPAIR - 1)
            def _():
                gath(i0 + 2, rows0, sem0a, sem0b)

            gwait(i0 + 1, rows1, sem1a, sem1b)
            pltpu.sync_copy(rows1, acc_sh.at[dstidx.at[i0 + 1]], add=True)
            return c

        lax.fori_loop(0, WPAIR, pair, 0)
        return carry

    lax.fori_loop(0, nwin, window, 0)
    plsc.subcore_barrier()
    pltpu.sync_copy(acc_sh.at[pl.ds(base_r, RPT)],
                    acc_out.at[cid, pl.ds(base_r, RPT)])


_DEG_SCRATCH = [
    pltpu.VMEM((NCHUNK, CH), jnp.int32),
    pltpu.VMEM((CH, D), jnp.float32),
    pltpu.VMEM_SHARED((ACC_N, D), jnp.float32),
]
_AGG_SCRATCH = [
    pltpu.VMEM((WCH, CH), jnp.int32),
    pltpu.VMEM((WCH, CH), jnp.int32),
    pltpu.VMEM((CH, D), jnp.float32),
    pltpu.VMEM((CH, D), jnp.float32),
    pltpu.VMEM_SHARED((ACC_N, D), jnp.float32),
    pltpu.SemaphoreType.DMA,
    pltpu.SemaphoreType.DMA,
    pltpu.SemaphoreType.DMA,
    pltpu.SemaphoreType.DMA,
]

_deg_kernel = pl.kernel(
    _deg_body,
    out_type=jax.ShapeDtypeStruct((NC, ACC_N, D), jnp.float32),
    mesh=_mesh,
    scratch_types=_DEG_SCRATCH,
)

_agg_kernel = pl.kernel(
    _agg_body,
    out_type=jax.ShapeDtypeStruct((NC, ACC_N, D), jnp.float32),
    mesh=_mesh,
    scratch_types=_AGG_SCRATCH,
)


def _dense1_body(x_ref, w_ref, d0_ref, d1_ref, y_ref, dinv_ref):
    deg = d0_ref[...] + d1_ref[...] + 1.0
    dinv = lax.rsqrt(deg)
    y_ref[...] = (x_ref[...] @ w_ref[...]) * dinv
    dinv_ref[...] = dinv


def _mid_body(a0_ref, a1_ref, yp_ref, dinv_ref, b_ref, g_ref, be_ref, w_ref,
              y_ref):
    dinv = dinv_ref[...]
    pre = (a0_ref[...] + a1_ref[...] + yp_ref[...]) * dinv + b_ref[...]
    mu = jnp.mean(pre, axis=0, keepdims=True)
    var = jnp.mean((pre - mu) ** 2, axis=0, keepdims=True)
    h = jnp.maximum((pre - mu) * lax.rsqrt(var + 1e-5) * g_ref[...]
                    + be_ref[...], 0.0)
    y_ref[...] = (h @ w_ref[...]) * dinv


def _final_body(a0_ref, a1_ref, yp_ref, dinv_ref, b_ref, out_ref):
    out_ref[...] = ((a0_ref[...] + a1_ref[...] + yp_ref[...]) * dinv_ref[...]
                    + b_ref[...])


_f32 = jnp.float32

_dense1 = pl.pallas_call(
    _dense1_body,
    out_shape=(jax.ShapeDtypeStruct((N, D), _f32),
               jax.ShapeDtypeStruct((N, 1), _f32)),
)

_mid = pl.pallas_call(
    _mid_body,
    out_shape=jax.ShapeDtypeStruct((N, D), _f32),
)

_final = pl.pallas_call(
    _final_body,
    out_shape=jax.ShapeDtypeStruct((N, D), _f32),
)


def kernel(x, edge_index, W1, b1, g1, be1, W2, b2, g2, be2, W3, b3):
    src = edge_index[0].astype(jnp.int32)
    dst = edge_index[1].astype(jnp.int32)
    pad = E_PAD - E
    src_p = jnp.concatenate([src, jnp.zeros((pad,), jnp.int32)])
    dst_p = jnp.concatenate([dst, jnp.full((pad,), N, jnp.int32)])
    src_c = src_p.reshape(E_PAD // CH, CH)
    dst_c = dst_p.reshape(E_PAD // CH, CH)
    dst_t = dst_p.reshape(NW, NCHUNK, CH)

    ones_rows = jnp.ones((CH, D), _f32)
    zrows = jnp.zeros((RPT, D), _f32)

    degp = _deg_kernel(dst_t, ones_rows, zrows)
    d0 = degp[0, :N, 0:1]
    d1 = degp[1, :N, 0:1]

    y1, dinv = _dense1(x, W1, d0, d1)
    acc = _agg_kernel(y1, src_c, dst_c, zrows)
    y2 = _mid(acc[0, :N], acc[1, :N], y1, dinv, b1.reshape(1, D),
              g1.reshape(1, D), be1.reshape(1, D), W2)
    acc = _agg_kernel(y2, src_c, dst_c, zrows)
    y3 = _mid(acc[0, :N], acc[1, :N], y2, dinv, b2.reshape(1, D),
              g2.reshape(1, D), be2.reshape(1, D), W3)
    acc = _agg_kernel(y3, src_c, dst_c, zrows)
    out = _final(acc[0, :N], acc[1, :N], y3, dinv, b3.reshape(1, D))
    return out


# speedup vs baseline: 1.0195x; 1.0195x over previous
import functools

import jax
import jax.numpy as jnp
from jax import lax
from jax.experimental import pallas as pl
from jax.experimental.pallas import tpu as pltpu, tpu_sc as plsc

N = 10000
D = 128
E = 320000

NC = 2
NS = 16
NW = NC * NS

CH = 128
NCHUNK = 80
E_PAD = NW * NCHUNK * CH
ACC_N = 10112
RPT = ACC_N // NS

_mesh = plsc.VectorSubcoreMesh(core_axis_name="c", subcore_axis_name="s",
                               num_cores=NC, num_subcores=NS)


def _deg_body(dst_hbm, ones_hbm, zrows_hbm, deg_out, dstidx, onesbuf, deg_sh):
    cid = lax.axis_index("c")
    sid = lax.axis_index("s")
    wid = cid * NS + sid
    base_r = sid * RPT
    pltpu.sync_copy(zrows_hbm, deg_sh.at[pl.ds(base_r, RPT)])
    pltpu.sync_copy(ones_hbm, onesbuf)
    pltpu.sync_copy(dst_hbm.at[wid], dstidx)
    plsc.subcore_barrier()

    def chunk(i, carry):
        pltpu.sync_copy(onesbuf, deg_sh.at[dstidx.at[i]], add=True)
        return carry

    lax.fori_loop(0, NCHUNK, chunk, 0)
    plsc.subcore_barrier()
    pltpu.sync_copy(deg_sh.at[pl.ds(base_r, RPT)],
                    deg_out.at[cid, pl.ds(base_r, RPT)])


WCH = 40
WPAIR = WCH // 2
SLOW_CORE = 1
C_SLOW = 80
C_FAST = 2 * NCHUNK - C_SLOW


def _agg_body(y_hbm, src_hbm, dst_hbm, zrows_hbm, acc_out,
              srcidx, dstidx, rows0, rows1, acc_sh, sem0, sem1):
    cid = lax.axis_index("c")
    sid = lax.axis_index("s")
    base_r = sid * RPT
    base_chunk = jnp.where(cid == SLOW_CORE, sid * C_SLOW,
                           NS * C_SLOW + sid * C_FAST)
    nwin = jnp.where(cid == SLOW_CORE, C_SLOW // WCH, C_FAST // WCH)
    pltpu.sync_copy(zrows_hbm, acc_sh.at[pl.ds(base_r, RPT)])
    plsc.subcore_barrier()

    def window(w, carry):
        cb = base_chunk + w * WCH
        pltpu.sync_copy(src_hbm.at[pl.ds(cb, WCH)], srcidx)
        pltpu.sync_copy(dst_hbm.at[pl.ds(cb, WCH)], dstidx)
        pltpu.async_copy(y_hbm.at[srcidx.at[0]], rows0, sem0)

        def pair(j, c):
            i0 = 2 * j
            pltpu.async_copy(y_hbm.at[srcidx.at[i0 + 1]], rows1, sem1)
            pltpu.make_async_copy(y_hbm.at[srcidx.at[i0]], rows0, sem0).wait()
            pltpu.sync_copy(rows0, acc_sh.at[dstidx.at[i0]], add=True)

            @pl.when(j < WPAIR - 1)
            def _():
                pltpu.async_copy(y_hbm.at[srcidx.at[i0 + 2]], rows0, sem0)

            pltpu.make_async_copy(y_hbm.at[srcidx.at[i0 + 1]], rows1,
                                  sem1).wait()
            pltpu.sync_copy(rows1, acc_sh.at[dstidx.at[i0 + 1]], add=True)
            return c

        lax.fori_loop(0, WPAIR, pair, 0)
        return carry

    lax.fori_loop(0, nwin, window, 0)
    plsc.subcore_barrier()
    pltpu.sync_copy(acc_sh.at[pl.ds(base_r, RPT)],
                    acc_out.at[cid, pl.ds(base_r, RPT)])


_DEG_SCRATCH = [
    pltpu.VMEM((NCHUNK, CH), jnp.int32),
    pltpu.VMEM((CH, D), jnp.float32),
    pltpu.VMEM_SHARED((ACC_N, D), jnp.float32),
]
_AGG_SCRATCH = [
    pltpu.VMEM((WCH, CH), jnp.int32),
    pltpu.VMEM((WCH, CH), jnp.int32),
    pltpu.VMEM((CH, D), jnp.float32),
    pltpu.VMEM((CH, D), jnp.float32),
    pltpu.VMEM_SHARED((ACC_N, D), jnp.float32),
    pltpu.SemaphoreType.DMA,
    pltpu.SemaphoreType.DMA,
]

_deg_kernel = pl.kernel(
    _deg_body,
    out_type=jax.ShapeDtypeStruct((NC, ACC_N, D), jnp.float32),
    mesh=_mesh,
    scratch_types=_DEG_SCRATCH,
)

_agg_kernel = pl.kernel(
    _agg_body,
    out_type=jax.ShapeDtypeStruct((NC, ACC_N, D), jnp.float32),
    mesh=_mesh,
    scratch_types=_AGG_SCRATCH,
)


def _dense1_body(x_ref, w_ref, d0_ref, d1_ref, y_ref, dinv_ref):
    deg = d0_ref[...] + d1_ref[...] + 1.0
    dinv = lax.rsqrt(deg)
    y_ref[...] = (x_ref[...] @ w_ref[...]) * dinv
    dinv_ref[...] = dinv


def _mid_body(a0_ref, a1_ref, yp_ref, dinv_ref, b_ref, g_ref, be_ref, w_ref,
              y_ref):
    dinv = dinv_ref[...]
    pre = (a0_ref[...] + a1_ref[...] + yp_ref[...]) * dinv + b_ref[...]
    mu = jnp.mean(pre, axis=0, keepdims=True)
    var = jnp.mean((pre - mu) ** 2, axis=0, keepdims=True)
    h = jnp.maximum((pre - mu) * lax.rsqrt(var + 1e-5) * g_ref[...]
                    + be_ref[...], 0.0)
    y_ref[...] = (h @ w_ref[...]) * dinv


def _final_body(a0_ref, a1_ref, yp_ref, dinv_ref, b_ref, out_ref):
    out_ref[...] = ((a0_ref[...] + a1_ref[...] + yp_ref[...]) * dinv_ref[...]
                    + b_ref[...])


_f32 = jnp.float32

_dense1 = pl.pallas_call(
    _dense1_body,
    out_shape=(jax.ShapeDtypeStruct((N, D), _f32),
               jax.ShapeDtypeStruct((N, 1), _f32)),
)

_mid = pl.pallas_call(
    _mid_body,
    out_shape=jax.ShapeDtypeStruct((N, D), _f32),
)

_final = pl.pallas_call(
    _final_body,
    out_shape=jax.ShapeDtypeStruct((N, D), _f32),
)


def kernel(x, edge_index, W1, b1, g1, be1, W2, b2, g2, be2, W3, b3):
    src = edge_index[0].astype(jnp.int32)
    dst = edge_index[1].astype(jnp.int32)
    pad = E_PAD - E
    src_p = jnp.concatenate([src, jnp.zeros((pad,), jnp.int32)])
    dst_p = jnp.concatenate([dst, jnp.full((pad,), N, jnp.int32)])
    src_c = src_p.reshape(E_PAD // CH, CH)
    dst_c = dst_p.reshape(E_PAD // CH, CH)
    dst_t = dst_p.reshape(NW, NCHUNK, CH)

    ones_rows = jnp.ones((CH, D), _f32)
    zrows = jnp.zeros((RPT, D), _f32)

    degp = _deg_kernel(dst_t, ones_rows, zrows)
    d0 = degp[0, :N, 0:1]
    d1 = degp[1, :N, 0:1]

    y1, dinv = _dense1(x, W1, d0, d1)
    acc = _agg_kernel(y1, src_c, dst_c, zrows)
    y2 = _mid(acc[0, :N], acc[1, :N], y1, dinv, b1.reshape(1, D),
              g1.reshape(1, D), be1.reshape(1, D), W2)
    acc = _agg_kernel(y2, src_c, dst_c, zrows)
    y3 = _mid(acc[0, :N], acc[1, :N], y2, dinv, b2.reshape(1, D),
              g2.reshape(1, D), be2.reshape(1, D), W3)
    acc = _agg_kernel(y3, src_c, dst_c, zrows)
    out = _final(acc[0, :N], acc[1, :N], y3, dinv, b3.reshape(1, D))
    return out
